# baseline (speedup 1.0000x reference)
import functools

import jax
import jax.numpy as jnp
from jax import lax
from jax._src.layout import Layout as _Layout
from jax._src.pjit import with_layout_constraint as _with_layout_constraint
from jax.experimental import pallas as pl
from jax.experimental.pallas import tpu as pltpu
from jax.experimental.pallas import tpu_sc as plsc

D = 64
SCALE = 8.0
NC = 2
NS = 16
L = 16
NW = NC * NS
CS = 128
NBUF = 4


@functools.lru_cache(maxsize=None)
def _make_emb(CH):
    G = CH // NBUF
    mesh = plsc.VectorSubcoreMesh(core_axis_name="c", subcore_axis_name="s")

    @functools.partial(
        pl.kernel,
        out_type=jax.ShapeDtypeStruct((NW, CH, CS // 2, 2 * D), jnp.float32),
        mesh=mesh,
        compiler_params=pltpu.CompilerParams(needs_layout_passes=False),
        scratch_types=[
            pltpu.VMEM((CH, CS), jnp.int32),
            pltpu.VMEM((NBUF, CS), jnp.int32),
            pltpu.VMEM((NBUF, CS, 2 * D), jnp.float32),
            pltpu.VMEM((NBUF, CS // 2, 2 * D), jnp.float32),
            pltpu.SemaphoreType.DMA((NBUF,)),
            pltpu.SemaphoreType.DMA((NBUF,)),
        ],
    )
    def emb(x_hbm, lut_hbm, out_hbm, idx_v, pidx_v, rows_g, rows_o,
            gsem, wsem):
        wid = lax.axis_index("s") * NC + lax.axis_index("c")
        pltpu.sync_copy(x_hbm.at[wid], idx_v)

        iota = lax.iota(jnp.int32, L)
        half = lax.shift_right_logical(iota, 1)
        pcbase = lax.shift_left(iota & 1, 6)
        bsplat = [jnp.full((L,), b, jnp.int32) for b in range(NBUF)]

        def prep_chunk(j, b):
            for k in range(CS // L):
                sl = pl.ds(k * L, L)
                pidx_v[b, sl] = lax.shift_right_logical(idx_v[j, sl], 1)
            pltpu.async_copy(lut_hbm.at[pidx_v.at[b]], rows_g.at[b],
                             gsem.at[b])

        def wait_gather(b):
            pltpu.make_async_copy(
                lut_hbm.at[pl.ds(0, CS)], rows_g.at[b], gsem.at[b]).wait()

        def fire_write(j, b):
            pltpu.async_copy(rows_o.at[b], out_hbm.at[wid, j], wsem.at[b])

        def wait_write(b):
            pltpu.make_async_copy(out_hbm.at[0, 0], rows_o.at[b],
                                  wsem.at[b]).wait()

        def scale(j, b):
            def block(t, carry):
                r0 = t * L
                idx16 = idx_v[j, pl.ds(r0, L)]
                srcc0 = lax.shift_left(idx16 & 1, 6)
                row16 = iota + r0
                pr16 = half + (r0 // 2)
                for c in range(D):
                    v = plsc.load_gather(
                        rows_g, [bsplat[b], row16, srcc0 + c])
                    plsc.store_scatter(
                        rows_o, [bsplat[b], pr16, pcbase + c], v * SCALE)
                return carry
            lax.fori_loop(0, CS // L, block, 0)

        def chunk_step(g, b, first, refire):
            j = g * NBUF + b
            wait_gather(b)
            if not first:
                wait_write(b)
            scale(j, b)
            fire_write(j, b)
            if refire:
                prep_chunk(j + NBUF, b)

        for b in range(NBUF):
            prep_chunk(b, b)
        for b in range(NBUF):
            chunk_step(0, b, True, True)

        def body(g, carry):
            for b in range(NBUF):
                chunk_step(g, b, False, True)
            return carry

        lax.fori_loop(1, G - 1, body, 0)

        for b in range(NBUF):
            chunk_step(G - 1, b, False, False)
        for b in range(NBUF):
            wait_write(b)

    return emb


def kernel(x, lut):
    S, Bt = x.shape
    B = S * Bt
    per_w = B // NW
    CH = per_w // CS
    xr = x.astype(jnp.int32).reshape(NW, CH, CS)
    lut2 = lut.reshape(lut.shape[0] // 2, 2 * D)
    out = _make_emb(CH)(xr, lut2)
    res = out.reshape(S, Bt, D)
    return res


# speedup vs baseline: 2.5530x; 2.5530x over previous
import functools

import jax
import jax.numpy as jnp
from jax import lax
from jax.experimental import pallas as pl
from jax.experimental.pallas import tpu as pltpu
from jax.experimental.pallas import tpu_sc as plsc

D = 64
SCALE = 8.0
NC = 2
NS = 16
L = 16
NW = NC * NS
CS = 128
NBUF = 4


@functools.lru_cache(maxsize=None)
def _make_emb(S, Bt):
    B = S * Bt
    per_w = B // NW
    CH = per_w // CS
    G = CH // NBUF
    mesh = plsc.VectorSubcoreMesh(core_axis_name="c", subcore_axis_name="s")

    @functools.partial(
        pl.kernel,
        out_type=jax.ShapeDtypeStruct((S, Bt, D), jnp.float32),
        mesh=mesh,
        compiler_params=pltpu.CompilerParams(use_tc_tiling_on_sc=False),
        scratch_types=[
            pltpu.VMEM((CH, CS), jnp.int32),
            pltpu.VMEM((NBUF, CS, D), jnp.float32),
            pltpu.VMEM((NBUF, CS, D), jnp.float32),
            pltpu.SemaphoreType.DMA((NBUF,)),
            pltpu.SemaphoreType.DMA((NBUF,)),
        ],
    )
    def emb(x_hbm, lut_hbm, out_hbm, idx_v, rows_g, rows_o, gsem, wsem):
        wid = lax.axis_index("s") * NC + lax.axis_index("c")
        pltpu.sync_copy(x_hbm.at[wid], idx_v)

        def fire_gather(j, b):
            pltpu.async_copy(lut_hbm.at[idx_v.at[j]], rows_g.at[b], gsem.at[b])

        def wait_gather(b):
            pltpu.make_async_copy(
                lut_hbm.at[pl.ds(0, CS)], rows_g.at[b], gsem.at[b]).wait()

        def fire_write(j, b):
            base = wid * per_w + j * CS
            t = base // Bt
            c0 = base % Bt
            pltpu.async_copy(rows_o.at[b], out_hbm.at[t, pl.ds(c0, CS)],
                             wsem.at[b])

        def wait_write(b):
            pltpu.make_async_copy(out_hbm.at[0, pl.ds(0, CS)], rows_o.at[b],
                                  wsem.at[b]).wait()

        def scale(b):
            def row(r, carry):
                for c in range(D // L):
                    sl = pl.ds(c * L, L)
                    rows_o[b, r, sl] = rows_g[b, r, sl] * SCALE
                return carry
            lax.fori_loop(0, CS, row, 0)

        def chunk_step(g, b, first, refire):
            j = g * NBUF + b
            wait_gather(b)
            if not first:
                wait_write(b)
            scale(b)
            if refire:
                fire_gather(j + NBUF, b)
            fire_write(j, b)

        for b in range(NBUF):
            fire_gather(b, b)
        for b in range(NBUF):
            chunk_step(0, b, True, True)

        def body(g, carry):
            for b in range(NBUF):
                chunk_step(g, b, False, True)
            return carry

        lax.fori_loop(1, G - 1, body, 0)

        for b in range(NBUF):
            chunk_step(G - 1, b, False, False)
        for b in range(NBUF):
            wait_write(b)

    return emb


def kernel(x, lut):
    S, Bt = x.shape
    B = S * Bt
    per_w = B // NW
    CH = per_w // CS
    xr = x.astype(jnp.int32).reshape(NW, CH, CS)
    return _make_emb(S, Bt)(xr, lut)


# speedup vs baseline: 2.5834x; 1.0119x over previous
import functools

import jax
import jax.numpy as jnp
from jax import lax
from jax.experimental import pallas as pl
from jax.experimental.pallas import tpu as pltpu
from jax.experimental.pallas import tpu_sc as plsc

D = 64
SCALE = 8.0
NC = 2
NS = 16
L = 16
NW = NC * NS
CS = 128
NBUF = 4
NOUT = 2


@functools.lru_cache(maxsize=None)
def _make_emb(S, Bt):
    B = S * Bt
    per_w = B // NW
    CH = per_w // CS
    G = CH // NBUF
    mesh = plsc.VectorSubcoreMesh(core_axis_name="c", subcore_axis_name="s")

    @functools.partial(
        pl.kernel,
        out_type=jax.ShapeDtypeStruct((S, Bt, D), jnp.float32),
        mesh=mesh,
        compiler_params=pltpu.CompilerParams(needs_layout_passes=False),
        scratch_types=[
            pltpu.VMEM((CH, CS), jnp.int32),
            pltpu.VMEM((NBUF, CS), jnp.int32),
            pltpu.VMEM((NBUF, CS, 2 * D), jnp.float32),
            pltpu.VMEM((NOUT, CS, D), jnp.float32),
            pltpu.SemaphoreType.DMA((NBUF,)),
            pltpu.SemaphoreType.DMA((NOUT,)),
        ],
    )
    def emb(x_hbm, lut_hbm, out_hbm, idx_v, pidx_v, rows_g, rows_o,
            gsem, wsem):
        wid = lax.axis_index("s") * NC + lax.axis_index("c")
        pltpu.sync_copy(x_hbm.at[wid], idx_v)

        lane = [jnp.full((L,), i, jnp.int32) for i in range(L)]

        def prep_chunk(j, b):
            for k in range(CS // L):
                sl = pl.ds(k * L, L)
                pidx_v[b, sl] = lax.shift_right_logical(idx_v[j, sl], 1)
            pltpu.async_copy(lut_hbm.at[pidx_v.at[b]], rows_g.at[b],
                             gsem.at[b])

        def wait_gather(b):
            pltpu.make_async_copy(
                lut_hbm.at[pl.ds(0, CS)], rows_g.at[b], gsem.at[b]).wait()

        def fire_write(j, ob):
            base = wid * per_w + j * CS
            t = base // Bt
            c0 = base % Bt
            pltpu.async_copy(rows_o.at[ob], out_hbm.at[t, pl.ds(c0, CS)],
                             wsem.at[ob])

        def wait_write(ob):
            pltpu.make_async_copy(out_hbm.at[0, pl.ds(0, CS)], rows_o.at[ob],
                                  wsem.at[ob]).wait()

        def scale(j, b, ob):
            def block(t, carry):
                r0 = t * L
                idx16 = idx_v[j, pl.ds(r0, L)]
                par16 = idx16 & 1
                for p in range(L):
                    r = r0 + p
                    m = jnp.take(par16, lane[p]) != 0
                    for c in range(D // L):
                        lo = rows_g[b, r, pl.ds(c * L, L)]
                        hi = rows_g[b, r, pl.ds(D + c * L, L)]
                        rows_o[ob, r, pl.ds(c * L, L)] = (
                            lax.select(m, hi, lo) * SCALE)
                return carry
            lax.fori_loop(0, CS // L, block, 0)

        def chunk_step(g, b, first, refire):
            j = g * NBUF + b
            ob = b % NOUT
            wait_gather(b)
            if not first:
                wait_write(ob)
            scale(j, b, ob)
            fire_write(j, ob)
            if refire:
                prep_chunk(j + NBUF, b)

        for b in range(NBUF):
            prep_chunk(b, b)
        for b in range(NBUF):
            chunk_step(0, b, b < NOUT, True)

        def body(g, carry):
            for b in range(NBUF):
                chunk_step(g, b, False, True)
            return carry

        lax.fori_loop(1, G - 1, body, 0)

        for b in range(NBUF):
            chunk_step(G - 1, b, False, False)
        for ob in range(NOUT):
            wait_write(ob)

    return emb


def kernel(x, lut):
    S, Bt = x.shape
    B = S * Bt
    per_w = B // NW
    CH = per_w // CS
    xr = x.astype(jnp.int32).reshape(NW, CH, CS)
    lut2 = lut.reshape(lut.shape[0] // 2, 2 * D)
    return _make_emb(S, Bt)(xr, lut2)


# speedup vs baseline: 2.9024x; 1.1235x over previous
import functools

import jax
import jax.numpy as jnp
from jax import lax
from jax.experimental import pallas as pl
from jax.experimental.pallas import tpu as pltpu
from jax.experimental.pallas import tpu_sc as plsc

D = 64
SCALE = 8.0
NC = 2
NS = 16
L = 16
NW = NC * NS
CS = 128
NBUF = 4
NOUT = 2


@functools.lru_cache(maxsize=None)
def _make_emb(S, Bt):
    B = S * Bt
    per_w = B // NW
    CH = per_w // CS
    G = CH // NBUF
    mesh = plsc.VectorSubcoreMesh(core_axis_name="c", subcore_axis_name="s")

    @functools.partial(
        pl.kernel,
        out_type=jax.ShapeDtypeStruct((S, Bt, D), jnp.float32),
        mesh=mesh,
        compiler_params=pltpu.CompilerParams(needs_layout_passes=False),
        scratch_types=[
            pltpu.VMEM((CH, CS), jnp.int32),
            pltpu.VMEM((NBUF, CS), jnp.int32),
            pltpu.VMEM((NBUF, CS, 2 * D), jnp.float32),
            pltpu.VMEM((NOUT, CS, D), jnp.float32),
            pltpu.SemaphoreType.DMA((NBUF,)),
            pltpu.SemaphoreType.DMA((NOUT,)),
        ],
    )
    def emb(x_hbm, lut_hbm, out_hbm, idx_v, pidx_v, rows_g, rows_o,
            gsem, wsem):
        wid = lax.axis_index("s") * NC + lax.axis_index("c")
        pltpu.sync_copy(x_hbm.at[wid], idx_v)

        lane = [jnp.full((L,), i, jnp.int32) for i in range(L)]

        def prep_chunk(j, b):
            for k in range(CS // L):
                sl = pl.ds(k * L, L)
                pidx_v[b, sl] = lax.shift_right_logical(idx_v[j, sl], 1)
            pltpu.async_copy(lut_hbm.at[pidx_v.at[b]], rows_g.at[b],
                             gsem.at[b])

        def wait_gather(b):
            pltpu.make_async_copy(
                lut_hbm.at[pl.ds(0, CS)], rows_g.at[b], gsem.at[b]).wait()

        def fire_write(j, ob):
            base = wid * per_w + j * CS
            t = base // Bt
            c0 = base % Bt
            pltpu.async_copy(rows_o.at[ob], out_hbm.at[t, pl.ds(c0, CS)],
                             wsem.at[ob])

        def wait_write(ob):
            pltpu.make_async_copy(out_hbm.at[0, pl.ds(0, CS)], rows_o.at[ob],
                                  wsem.at[ob]).wait()

        def scale(j, b, ob):
            def block(t, carry):
                r0 = t * L
                idx16 = idx_v[j, pl.ds(r0, L)]
                par16 = idx16 & 1
                for p in range(L):
                    r = r0 + p
                    m = jnp.take(par16, lane[p]) != 0
                    for c in range(D // L):
                        lo = rows_g[b, r, pl.ds(c * L, L)]
                        hi = rows_g[b, r, pl.ds(D + c * L, L)]
                        rows_o[ob, r, pl.ds(c * L, L)] = (
                            lax.select(m, hi, lo) * SCALE)
                return carry
            lax.fori_loop(0, CS // L, block, 0)

        def chunk_step(g, b, first, refire):
            j = g * NBUF + b
            ob = b % NOUT
            wait_gather(b)
            if not first:
                wait_write(ob)
            scale(j, b, ob)
            fire_write(j, ob)
            if refire:
                prep_chunk(j + NBUF, b)

        for b in range(NBUF):
            prep_chunk(b, b)
        for b in range(NBUF):
            chunk_step(0, b, b < NOUT, True)

        def body(g, carry):
            for b in range(NBUF):
                chunk_step(g, b, False, True)
            return carry

        lax.fori_loop(1, G - 1, body, 0)

        for b in range(NBUF):
            chunk_step(G - 1, b, False, False)
        for ob in range(NOUT):
            wait_write(ob)

    return emb


def kernel(x, lut):
    S, Bt = x.shape
    B = S * Bt
    per_w = B // NW
    CH = per_w // CS
    xr = x.astype(jnp.int32).reshape(NW, CH, CS)
    lut2 = lut.reshape(lut.shape[0] // 2, 2 * D)
    out = _make_emb(S, Bt)(xr, lut2)
    return jax.lax.optimization_barrier(out)
